# baseline (speedup 1.0000x reference)
import functools

import jax
import jax.numpy as jnp
from jax import lax
from jax.experimental import pallas as pl
from jax.experimental.pallas import tpu as pltpu
from jax.experimental.pallas import tpu_sc as plsc

_N = 10000
_E = 320000
_G = 64
_NC = 2
_NS = 16
_NW = _NC * _NS
_CH = 128
_EP = 327680
_EPW = _EP // _NW
_NIT = _EPW // _CH
_NP = 10240
_RPT = _NP // _NS
_ZR = 80

_BR = 2048
_NB = 5

_mesh = plsc.VectorSubcoreMesh(core_axis_name="c", subcore_axis_name="s")
_sc_params = pltpu.CompilerParams(use_tc_tiling_on_sc=False)


def _sc_degree(e1, e2, e3):
    outs = tuple(jax.ShapeDtypeStruct((_NC, _NP), jnp.float32) for _ in range(3))
    scratch = [
        pltpu.VMEM((_NIT, _CH), jnp.int32),
        pltpu.VMEM((_CH,), jnp.float32),
        pltpu.VMEM((_RPT,), jnp.float32),
        pltpu.VMEM_SHARED((_NP,), jnp.float32),
        pltpu.VMEM_SHARED((_NP,), jnp.float32),
        pltpu.VMEM_SHARED((_NP,), jnp.float32),
    ]

    @functools.partial(pl.kernel, out_type=outs, mesh=_mesh,
                       scratch_types=scratch, compiler_params=_sc_params)
    def k(e1r, e2r, e3r, o1, o2, o3, dstv, onev, zv, a1, a2, a3):
        c = lax.axis_index("c")
        s = lax.axis_index("s")
        wid = c * _NS + s

        def fill_one(i, _):
            onev[pl.ds(i * 16, 16)] = jnp.ones((16,), jnp.float32)
            return 0
        lax.fori_loop(0, _CH // 16, fill_one, 0)

        def fill_zero(i, _):
            zv[pl.ds(i * 16, 16)] = jnp.zeros((16,), jnp.float32)
            return 0
        lax.fori_loop(0, _RPT // 16, fill_zero, 0)

        for a in (a1, a2, a3):
            pltpu.sync_copy(zv, a.at[pl.ds(s * _RPT, _RPT)])
        plsc.subcore_barrier()

        for er, a in ((e1r, a1), (e2r, a2), (e3r, a3)):
            pltpu.sync_copy(er.at[1, wid], dstv)

            def it(i, _, a=a):
                pltpu.sync_copy(onev, a.at[dstv.at[i]], add=True)
                return 0
            lax.fori_loop(0, _NIT, it, 0)
        plsc.subcore_barrier()

        for o, a in ((o1, a1), (o2, a2), (o3, a3)):
            pltpu.sync_copy(a.at[pl.ds(s * _RPT, _RPT)],
                            o.at[c, pl.ds(s * _RPT, _RPT)])

    return k(e1, e2, e3)


def _sc_scatter(F, y1, y2, y3, e1, e2, e3):
    outs = tuple(jax.ShapeDtypeStruct((_NC, _NP, F), jnp.float32)
                 for _ in range(3))
    scratch = [
        pltpu.VMEM((_NIT, _CH), jnp.int32),
        pltpu.VMEM((_NIT, _CH), jnp.int32),
        pltpu.VMEM((_CH, F), jnp.float32),
        pltpu.VMEM((_CH, F), jnp.float32),
        pltpu.VMEM((_ZR, F), jnp.float32),
        pltpu.SemaphoreType.DMA,
        pltpu.SemaphoreType.DMA,
        pltpu.SemaphoreType.DMA,
        pltpu.SemaphoreType.DMA,
        pltpu.VMEM_SHARED((_NP, F), jnp.float32),
    ]

    @functools.partial(pl.kernel, out_type=outs, mesh=_mesh,
                       scratch_types=scratch, compiler_params=_sc_params)
    def k(y1r, y2r, y3r, e1r, e2r, e3r, o1, o2, o3,
          srcv, dstv, rows0, rows1, zrow, g0, g1, s0, s1, acc):
        c = lax.axis_index("c")
        s = lax.axis_index("s")
        wid = c * _NS + s

        def zfill(r, _):
            for j in range(F // 16):
                zrow[r, pl.ds(j * 16, 16)] = jnp.zeros((16,), jnp.float32)
            return 0
        lax.fori_loop(0, _ZR, zfill, 0)

        def zero_own_slice():
            def cz(j, _):
                pltpu.sync_copy(zrow, acc.at[pl.ds(s * _RPT + j * _ZR, _ZR)])
                return 0
            lax.fori_loop(0, _RPT // _ZR, cz, 0)

        zero_own_slice()
        plsc.subcore_barrier()

        for yr, er, o in ((y1r, e1r, o1), (y2r, e2r, o2), (y3r, e3r, o3)):
            pltpu.sync_copy(er.at[0, wid], srcv)
            pltpu.sync_copy(er.at[1, wid], dstv)

            pltpu.async_copy(yr.at[srcv.at[0]], rows0, g0)

            def it(j, _, yr=yr):
                i0 = 2 * j
                pltpu.make_async_copy(yr.at[srcv.at[i0]], rows0, g0).wait()
                pltpu.async_copy(rows0, acc.at[dstv.at[i0]], s0, add=True)

                @pl.when(j > 0)
                def _s1_done():
                    pltpu.make_async_copy(
                        rows1, acc.at[dstv.at[i0]], s1).wait()
                pltpu.async_copy(yr.at[srcv.at[i0 + 1]], rows1, g1)

                pltpu.make_async_copy(yr.at[srcv.at[i0]], rows1, g1).wait()
                pltpu.async_copy(rows1, acc.at[dstv.at[i0 + 1]], s1, add=True)
                pltpu.make_async_copy(rows0, acc.at[dstv.at[i0]], s0).wait()

                @pl.when(j < _NIT // 2 - 1)
                def _next():
                    pltpu.async_copy(yr.at[srcv.at[i0 + 2]], rows0, g0)
                return 0
            lax.fori_loop(0, _NIT // 2, it, 0)
            pltpu.make_async_copy(rows1, acc.at[dstv.at[0]], s1).wait()
            plsc.subcore_barrier()
            pltpu.sync_copy(acc.at[pl.ds(s * _RPT, _RPT)],
                            o.at[c, pl.ds(s * _RPT, _RPT)])
            if o is not o3:
                zero_own_slice()
                plsc.subcore_barrier()

    return k(y1, y2, y3, e1, e2, e3)


def _tc_y(x, degt, W1):
    def body(x_ref, dg_ref, w_ref, y_ref):
        dinv = lax.rsqrt(dg_ref[...].sum(axis=1, keepdims=True) + 1.0)
        xw = jnp.dot(x_ref[...], w_ref[...],
                     preferred_element_type=jnp.float32)
        y_ref[...] = xw * dinv

    return pl.pallas_call(
        body, grid=(_NB,),
        in_specs=[
            pl.BlockSpec((_BR, 128), lambda i: (i, 0)),
            pl.BlockSpec((_BR, 2), lambda i: (i, 0)),
            pl.BlockSpec((128, 64), lambda i: (0, 0)),
        ],
        out_specs=pl.BlockSpec((_BR, 64), lambda i: (i, 0)),
        out_shape=jax.ShapeDtypeStruct((_N, 64), jnp.float32),
    )(x, degt, W1)


def _tc_mid(p, y, degt, W2, b1):
    def body(p_ref, y_ref, dg_ref, w_ref, b_ref, o_ref):
        dinv = lax.rsqrt(dg_ref[...].sum(axis=1, keepdims=True) + 1.0)
        pre = p_ref[0] + p_ref[1] + y_ref[...]
        h = jnp.maximum(pre * dinv + b_ref[...], 0.0)
        o_ref[...] = jnp.dot(h, w_ref[...],
                             preferred_element_type=jnp.float32) * dinv

    return pl.pallas_call(
        body, grid=(_NB,),
        in_specs=[
            pl.BlockSpec((2, _BR, 64), lambda i: (0, i, 0)),
            pl.BlockSpec((_BR, 64), lambda i: (i, 0)),
            pl.BlockSpec((_BR, 2), lambda i: (i, 0)),
            pl.BlockSpec((64, 32), lambda i: (0, 0)),
            pl.BlockSpec((1, 64), lambda i: (0, 0)),
        ],
        out_specs=pl.BlockSpec((_BR, 32), lambda i: (i, 0)),
        out_shape=jax.ShapeDtypeStruct((_N, 32), jnp.float32),
    )(p, y, degt, W2, b1)


def _tc_head(p2, y2, degt, batch3, b2, fW1, fb1, fW2, fb2):
    def body(p_ref, y_ref, dg_ref, bt_ref, b2_ref,
             w1_ref, c1_ref, w2_ref, c2_ref, o_ref, acc):
        i = pl.program_id(0)

        @pl.when(i == 0)
        def _init():
            acc[...] = jnp.zeros((_G, 64), jnp.float32)

        dinv = lax.rsqrt(dg_ref[...].sum(axis=1, keepdims=True) + 1.0)
        o2 = (p_ref[0] + p_ref[1] + y_ref[...]) * dinv + b2_ref[...]
        rows = lax.broadcasted_iota(jnp.int32, (_BR, 1), 0) + i * _BR
        valid = (rows < _N).astype(jnp.float32)
        o2 = jnp.where(rows < _N, o2, 0.0)
        b = bt_ref[0, 0, :]
        P = (b[:, None] == lax.broadcasted_iota(jnp.int32, (_BR, _G), 1)
             ).astype(jnp.float32)
        ext = jnp.concatenate(
            [o2, valid, jnp.zeros((_BR, 31), jnp.float32)], axis=1)
        acc[...] += lax.dot_general(P, ext, (((0,), (0,)), ((), ())),
                                    preferred_element_type=jnp.float32)

        @pl.when(i == _NB - 1)
        def _fin():
            a = acc[...]
            pooled = a[:, :32] / jnp.maximum(a[:, 32:33], 1.0)
            z = jnp.maximum(
                jnp.dot(pooled, w1_ref[...],
                        preferred_element_type=jnp.float32) + c1_ref[...], 0.0)
            o_ref[...] = jnp.dot(z, w2_ref[...],
                                 preferred_element_type=jnp.float32) + c2_ref[...]

    return pl.pallas_call(
        body, grid=(_NB,),
        in_specs=[
            pl.BlockSpec((2, _BR, 32), lambda i: (0, i, 0)),
            pl.BlockSpec((_BR, 32), lambda i: (i, 0)),
            pl.BlockSpec((_BR, 2), lambda i: (i, 0)),
            pl.BlockSpec((1, 1, _BR), lambda i: (i, 0, 0)),
            pl.BlockSpec((1, 32), lambda i: (0, 0)),
            pl.BlockSpec((32, 32), lambda i: (0, 0)),
            pl.BlockSpec((1, 32), lambda i: (0, 0)),
            pl.BlockSpec((32, 32), lambda i: (0, 0)),
            pl.BlockSpec((1, 32), lambda i: (0, 0)),
        ],
        out_specs=pl.BlockSpec((_G, 32), lambda i: (0, 0)),
        out_shape=jax.ShapeDtypeStruct((_G, 32), jnp.float32),
        scratch_shapes=[pltpu.VMEM((_G, 64), jnp.float32)],
    )(p2, y2, degt, batch3, b2, fW1, fb1, fW2, fb2)


def kernel(x1, edge_index1, batch1, x2, edge_index2, batch2,
           x3, edge_index3, batch3, W1, b1, W2, b2, fW1, fb1, fW2, fb2):
    npad = _EP - _E
    src_pad = (jnp.arange(npad, dtype=jnp.int32) % _N)
    dst_pad = _N + (jnp.arange(npad, dtype=jnp.int32) % (_NP - _N))
    epad = jnp.stack([src_pad, dst_pad])

    e1 = jnp.concatenate([edge_index1, epad], 1).reshape(2, _NW, _NIT, _CH)
    e2 = jnp.concatenate([edge_index2, epad], 1).reshape(2, _NW, _NIT, _CH)
    e3 = jnp.concatenate([edge_index3, epad], 1).reshape(2, _NW, _NIT, _CH)

    d1, d2, d3 = _sc_degree(e1, e2, e3)
    degt1, degt2, degt3 = d1.T, d2.T, d3.T

    y1 = _tc_y(x1, degt1, W1)
    y2 = _tc_y(x2, degt2, W1)
    y3 = _tc_y(x3, degt3, W1)

    p1, p2, p3 = _sc_scatter(64, y1, y2, y3, e1, e2, e3)

    b1r = b1.reshape(1, 64)
    z1 = _tc_mid(p1, y1, degt1, W2, b1r)
    z2 = _tc_mid(p2, y2, degt2, W2, b1r)
    z3 = _tc_mid(p3, y3, degt3, W2, b1r)

    q1, q2, q3 = _sc_scatter(32, z1, z2, z3, e1, e2, e3)

    pad = _NP - _N
    bt1 = jnp.pad(batch1, (0, pad), constant_values=_G).reshape(_NB, 1, _BR)
    bt2 = jnp.pad(batch2, (0, pad), constant_values=_G).reshape(_NB, 1, _BR)
    bt3 = jnp.pad(batch3, (0, pad), constant_values=_G).reshape(_NB, 1, _BR)
    b2r = b2.reshape(1, 32)
    fb1r = fb1.reshape(1, 32)
    fb2r = fb2.reshape(1, 32)

    o1 = _tc_head(q1, z1, degt1, bt1, b2r, fW1, fb1r, fW2, fb2r)
    o2 = _tc_head(q2, z2, degt2, bt2, b2r, fW1, fb1r, fW2, fb2r)
    o3 = _tc_head(q3, z3, degt3, bt3, b2r, fW1, fb1r, fW2, fb2r)
    return (o1, o2, o3)


# speedup vs baseline: 1.2181x; 1.2181x over previous
import functools

import jax
import jax.numpy as jnp
from jax import lax
from jax.experimental import pallas as pl
from jax.experimental.pallas import tpu as pltpu
from jax.experimental.pallas import tpu_sc as plsc

_N = 10000
_E = 320000
_G = 64
_NC = 2
_NS = 16
_NW = _NC * _NS
_CH = 128
_EP = 327680
_EPW = _EP // _NW
_NIT = _EPW // _CH
_NP = 10240
_RPT = _NP // _NS
_ZR = 80

_BR = 2048
_NB = 5

_mesh = plsc.VectorSubcoreMesh(core_axis_name="c", subcore_axis_name="s")
_sc_params = pltpu.CompilerParams(use_tc_tiling_on_sc=False)


def _sc_degree(e1, e2, e3):
    outs = tuple(jax.ShapeDtypeStruct((_NC, _NP), jnp.float32) for _ in range(3))
    scratch = [
        pltpu.VMEM((_NIT, _CH), jnp.int32),
        pltpu.VMEM((_CH,), jnp.float32),
        pltpu.VMEM((_RPT,), jnp.float32),
        pltpu.VMEM_SHARED((_NP,), jnp.float32),
        pltpu.VMEM_SHARED((_NP,), jnp.float32),
        pltpu.VMEM_SHARED((_NP,), jnp.float32),
    ]

    @functools.partial(pl.kernel, out_type=outs, mesh=_mesh,
                       scratch_types=scratch, compiler_params=_sc_params)
    def k(e1r, e2r, e3r, o1, o2, o3, dstv, onev, zv, a1, a2, a3):
        c = lax.axis_index("c")
        s = lax.axis_index("s")
        wid = c * _NS + s

        def fill_one(i, _):
            onev[pl.ds(i * 16, 16)] = jnp.ones((16,), jnp.float32)
            return 0
        lax.fori_loop(0, _CH // 16, fill_one, 0)

        def fill_zero(i, _):
            zv[pl.ds(i * 16, 16)] = jnp.zeros((16,), jnp.float32)
            return 0
        lax.fori_loop(0, _RPT // 16, fill_zero, 0)

        for a in (a1, a2, a3):
            pltpu.sync_copy(zv, a.at[pl.ds(s * _RPT, _RPT)])
        plsc.subcore_barrier()

        for er, a in ((e1r, a1), (e2r, a2), (e3r, a3)):
            pltpu.sync_copy(er.at[1, wid], dstv)

            def it(i, _, a=a):
                pltpu.sync_copy(onev, a.at[dstv.at[i]], add=True)
                return 0
            lax.fori_loop(0, _NIT, it, 0)
        plsc.subcore_barrier()

        for o, a in ((o1, a1), (o2, a2), (o3, a3)):
            pltpu.sync_copy(a.at[pl.ds(s * _RPT, _RPT)],
                            o.at[c, pl.ds(s * _RPT, _RPT)])

    return k(e1, e2, e3)


def _sc_scatter(F, y1, y2, y3, e1, e2, e3):
    outs = tuple(jax.ShapeDtypeStruct((_NC, _NP, F), jnp.float32)
                 for _ in range(3))
    scratch = [
        pltpu.VMEM((_NIT, _CH), jnp.int32),
        pltpu.VMEM((_NIT, _CH), jnp.int32),
        pltpu.VMEM((_CH, F), jnp.float32),
        pltpu.VMEM((_CH, F), jnp.float32),
        pltpu.VMEM((_ZR, F), jnp.float32),
        pltpu.SemaphoreType.DMA,
        pltpu.SemaphoreType.DMA,
        pltpu.SemaphoreType.DMA,
        pltpu.SemaphoreType.DMA,
        pltpu.VMEM_SHARED((_NP, F), jnp.float32),
        pltpu.VMEM_SHARED((_N, F), jnp.float32),
    ]

    @functools.partial(pl.kernel, out_type=outs, mesh=_mesh,
                       scratch_types=scratch, compiler_params=_sc_params)
    def k(y1r, y2r, y3r, e1r, e2r, e3r, o1, o2, o3,
          srcv, dstv, rows0, rows1, zrow, g0, g1, s0, s1, acc, ysh):
        c = lax.axis_index("c")
        s = lax.axis_index("s")
        wid = c * _NS + s

        def zfill(r, _):
            for j in range(F // 16):
                zrow[r, pl.ds(j * 16, 16)] = jnp.zeros((16,), jnp.float32)
            return 0
        lax.fori_loop(0, _ZR, zfill, 0)

        def zero_own_slice():
            def cz(j, _):
                pltpu.sync_copy(zrow, acc.at[pl.ds(s * _RPT + j * _ZR, _ZR)])
                return 0
            lax.fori_loop(0, _RPT // _ZR, cz, 0)

        zero_own_slice()
        plsc.subcore_barrier()

        _SR = _N // _NS
        for yr, er, o in ((y1r, e1r, o1), (y2r, e2r, o2), (y3r, e3r, o3)):
            pltpu.sync_copy(er.at[0, wid], srcv)
            pltpu.sync_copy(er.at[1, wid], dstv)
            pltpu.sync_copy(yr.at[pl.ds(s * _SR, _SR)],
                            ysh.at[pl.ds(s * _SR, _SR)])
            plsc.subcore_barrier()

            pltpu.async_copy(ysh.at[srcv.at[0]], rows0, g0)

            def it(j, _):
                i0 = 2 * j
                pltpu.make_async_copy(ysh.at[srcv.at[i0]], rows0, g0).wait()
                pltpu.async_copy(rows0, acc.at[dstv.at[i0]], s0, add=True)

                @pl.when(j > 0)
                def _s1_done():
                    pltpu.make_async_copy(
                        rows1, acc.at[dstv.at[i0]], s1).wait()
                pltpu.async_copy(ysh.at[srcv.at[i0 + 1]], rows1, g1)

                pltpu.make_async_copy(ysh.at[srcv.at[i0]], rows1, g1).wait()
                pltpu.async_copy(rows1, acc.at[dstv.at[i0 + 1]], s1, add=True)
                pltpu.make_async_copy(rows0, acc.at[dstv.at[i0]], s0).wait()

                @pl.when(j < _NIT // 2 - 1)
                def _next():
                    pltpu.async_copy(ysh.at[srcv.at[i0 + 2]], rows0, g0)
                return 0
            lax.fori_loop(0, _NIT // 2, it, 0)
            pltpu.make_async_copy(rows1, acc.at[dstv.at[0]], s1).wait()
            plsc.subcore_barrier()
            pltpu.sync_copy(acc.at[pl.ds(s * _RPT, _RPT)],
                            o.at[c, pl.ds(s * _RPT, _RPT)])
            if o is not o3:
                zero_own_slice()
                plsc.subcore_barrier()

    return k(y1, y2, y3, e1, e2, e3)


def _tc_y(x, degt, W1):
    def body(x_ref, dg_ref, w_ref, y_ref):
        dinv = lax.rsqrt(dg_ref[...].sum(axis=1, keepdims=True) + 1.0)
        xw = jnp.dot(x_ref[...], w_ref[...],
                     preferred_element_type=jnp.float32)
        y_ref[...] = xw * dinv

    return pl.pallas_call(
        body, grid=(_NB,),
        in_specs=[
            pl.BlockSpec((_BR, 128), lambda i: (i, 0)),
            pl.BlockSpec((_BR, 2), lambda i: (i, 0)),
            pl.BlockSpec((128, 64), lambda i: (0, 0)),
        ],
        out_specs=pl.BlockSpec((_BR, 64), lambda i: (i, 0)),
        out_shape=jax.ShapeDtypeStruct((_N, 64), jnp.float32),
    )(x, degt, W1)


def _tc_mid(p, y, degt, W2, b1):
    def body(p_ref, y_ref, dg_ref, w_ref, b_ref, o_ref):
        dinv = lax.rsqrt(dg_ref[...].sum(axis=1, keepdims=True) + 1.0)
        pre = p_ref[0] + p_ref[1] + y_ref[...]
        h = jnp.maximum(pre * dinv + b_ref[...], 0.0)
        o_ref[...] = jnp.dot(h, w_ref[...],
                             preferred_element_type=jnp.float32) * dinv

    return pl.pallas_call(
        body, grid=(_NB,),
        in_specs=[
            pl.BlockSpec((2, _BR, 64), lambda i: (0, i, 0)),
            pl.BlockSpec((_BR, 64), lambda i: (i, 0)),
            pl.BlockSpec((_BR, 2), lambda i: (i, 0)),
            pl.BlockSpec((64, 32), lambda i: (0, 0)),
            pl.BlockSpec((1, 64), lambda i: (0, 0)),
        ],
        out_specs=pl.BlockSpec((_BR, 32), lambda i: (i, 0)),
        out_shape=jax.ShapeDtypeStruct((_N, 32), jnp.float32),
    )(p, y, degt, W2, b1)


def _tc_head(p2, y2, degt, batch3, b2, fW1, fb1, fW2, fb2):
    def body(p_ref, y_ref, dg_ref, bt_ref, b2_ref,
             w1_ref, c1_ref, w2_ref, c2_ref, o_ref, acc):
        i = pl.program_id(0)

        @pl.when(i == 0)
        def _init():
            acc[...] = jnp.zeros((_G, 64), jnp.float32)

        dinv = lax.rsqrt(dg_ref[...].sum(axis=1, keepdims=True) + 1.0)
        o2 = (p_ref[0] + p_ref[1] + y_ref[...]) * dinv + b2_ref[...]
        rows = lax.broadcasted_iota(jnp.int32, (_BR, 1), 0) + i * _BR
        valid = (rows < _N).astype(jnp.float32)
        o2 = jnp.where(rows < _N, o2, 0.0)
        b = bt_ref[0, 0, :]
        P = (b[:, None] == lax.broadcasted_iota(jnp.int32, (_BR, _G), 1)
             ).astype(jnp.float32)
        ext = jnp.concatenate(
            [o2, valid, jnp.zeros((_BR, 31), jnp.float32)], axis=1)
        acc[...] += lax.dot_general(P, ext, (((0,), (0,)), ((), ())),
                                    preferred_element_type=jnp.float32)

        @pl.when(i == _NB - 1)
        def _fin():
            a = acc[...]
            pooled = a[:, :32] / jnp.maximum(a[:, 32:33], 1.0)
            z = jnp.maximum(
                jnp.dot(pooled, w1_ref[...],
                        preferred_element_type=jnp.float32) + c1_ref[...], 0.0)
            o_ref[...] = jnp.dot(z, w2_ref[...],
                                 preferred_element_type=jnp.float32) + c2_ref[...]

    return pl.pallas_call(
        body, grid=(_NB,),
        in_specs=[
            pl.BlockSpec((2, _BR, 32), lambda i: (0, i, 0)),
            pl.BlockSpec((_BR, 32), lambda i: (i, 0)),
            pl.BlockSpec((_BR, 2), lambda i: (i, 0)),
            pl.BlockSpec((1, 1, _BR), lambda i: (i, 0, 0)),
            pl.BlockSpec((1, 32), lambda i: (0, 0)),
            pl.BlockSpec((32, 32), lambda i: (0, 0)),
            pl.BlockSpec((1, 32), lambda i: (0, 0)),
            pl.BlockSpec((32, 32), lambda i: (0, 0)),
            pl.BlockSpec((1, 32), lambda i: (0, 0)),
        ],
        out_specs=pl.BlockSpec((_G, 32), lambda i: (0, 0)),
        out_shape=jax.ShapeDtypeStruct((_G, 32), jnp.float32),
        scratch_shapes=[pltpu.VMEM((_G, 64), jnp.float32)],
    )(p2, y2, degt, batch3, b2, fW1, fb1, fW2, fb2)


def kernel(x1, edge_index1, batch1, x2, edge_index2, batch2,
           x3, edge_index3, batch3, W1, b1, W2, b2, fW1, fb1, fW2, fb2):
    npad = _EP - _E
    src_pad = (jnp.arange(npad, dtype=jnp.int32) % _N)
    dst_pad = _N + (jnp.arange(npad, dtype=jnp.int32) % (_NP - _N))
    epad = jnp.stack([src_pad, dst_pad])

    e1 = jnp.concatenate([edge_index1, epad], 1).reshape(2, _NW, _NIT, _CH)
    e2 = jnp.concatenate([edge_index2, epad], 1).reshape(2, _NW, _NIT, _CH)
    e3 = jnp.concatenate([edge_index3, epad], 1).reshape(2, _NW, _NIT, _CH)

    d1, d2, d3 = _sc_degree(e1, e2, e3)
    degt1, degt2, degt3 = d1.T, d2.T, d3.T

    y1 = _tc_y(x1, degt1, W1)
    y2 = _tc_y(x2, degt2, W1)
    y3 = _tc_y(x3, degt3, W1)

    p1, p2, p3 = _sc_scatter(64, y1, y2, y3, e1, e2, e3)

    b1r = b1.reshape(1, 64)
    z1 = _tc_mid(p1, y1, degt1, W2, b1r)
    z2 = _tc_mid(p2, y2, degt2, W2, b1r)
    z3 = _tc_mid(p3, y3, degt3, W2, b1r)

    q1, q2, q3 = _sc_scatter(32, z1, z2, z3, e1, e2, e3)

    pad = _NP - _N
    bt1 = jnp.pad(batch1, (0, pad), constant_values=_G).reshape(_NB, 1, _BR)
    bt2 = jnp.pad(batch2, (0, pad), constant_values=_G).reshape(_NB, 1, _BR)
    bt3 = jnp.pad(batch3, (0, pad), constant_values=_G).reshape(_NB, 1, _BR)
    b2r = b2.reshape(1, 32)
    fb1r = fb1.reshape(1, 32)
    fb2r = fb2.reshape(1, 32)

    o1 = _tc_head(q1, z1, degt1, bt1, b2r, fW1, fb1r, fW2, fb2r)
    o2 = _tc_head(q2, z2, degt2, bt2, b2r, fW1, fb1r, fW2, fb2r)
    o3 = _tc_head(q3, z3, degt3, bt3, b2r, fW1, fb1r, fW2, fb2r)
    return (o1, o2, o3)


# speedup vs baseline: 1.5256x; 1.2524x over previous
import functools

import jax
import jax.numpy as jnp
from jax import lax
from jax.experimental import pallas as pl
from jax.experimental.pallas import tpu as pltpu
from jax.experimental.pallas import tpu_sc as plsc

_N = 10000
_E = 320000
_G = 64
_NC = 2
_NS = 16
_NW = _NC * _NS
_CH = 128
_EP = 327680
_EPW = _EP // _NW
_NIT = _EPW // _CH
_NP = 10240
_RPT = _NP // _NS
_ZR = 80

_BR = 2048
_NB = 5

_mesh = plsc.VectorSubcoreMesh(core_axis_name="c", subcore_axis_name="s")
_sc_params = pltpu.CompilerParams(use_tc_tiling_on_sc=False)


def _sc_degree(e):
    out = jax.ShapeDtypeStruct((_NC, _NP), jnp.float32)
    scratch = [
        pltpu.VMEM((_NIT, _CH), jnp.int32),
        pltpu.VMEM((_CH,), jnp.float32),
        pltpu.VMEM((_RPT,), jnp.float32),
        pltpu.VMEM_SHARED((_NP,), jnp.float32),
    ]

    @functools.partial(pl.kernel, out_type=out, mesh=_mesh,
                       scratch_types=scratch, compiler_params=_sc_params)
    def k(er, o, dstv, onev, zv, a):
        c = lax.axis_index("c")
        s = lax.axis_index("s")
        wid = c * _NS + s

        def fill_one(i, _):
            onev[pl.ds(i * 16, 16)] = jnp.ones((16,), jnp.float32)
            return 0
        lax.fori_loop(0, _CH // 16, fill_one, 0)

        def fill_zero(i, _):
            zv[pl.ds(i * 16, 16)] = jnp.zeros((16,), jnp.float32)
            return 0
        lax.fori_loop(0, _RPT // 16, fill_zero, 0)

        pltpu.sync_copy(zv, a.at[pl.ds(s * _RPT, _RPT)])
        pltpu.sync_copy(er.at[1, wid], dstv)
        plsc.subcore_barrier()

        def it(i, _):
            pltpu.sync_copy(onev, a.at[dstv.at[i]], add=True)
            return 0
        lax.fori_loop(0, _NIT, it, 0)
        plsc.subcore_barrier()

        pltpu.sync_copy(a.at[pl.ds(s * _RPT, _RPT)],
                        o.at[c, pl.ds(s * _RPT, _RPT)])

    return k(e)


def _sc_scatter(F, y, e):
    out = jax.ShapeDtypeStruct((_NC, _NP, F), jnp.float32)
    scratch = [
        pltpu.VMEM((_NIT, _CH), jnp.int32),
        pltpu.VMEM((_NIT, _CH), jnp.int32),
        pltpu.VMEM((_CH, F), jnp.float32),
        pltpu.VMEM((_CH, F), jnp.float32),
        pltpu.VMEM((_ZR, F), jnp.float32),
        pltpu.SemaphoreType.DMA,
        pltpu.SemaphoreType.DMA,
        pltpu.SemaphoreType.DMA,
        pltpu.SemaphoreType.DMA,
        pltpu.VMEM_SHARED((_NP, F), jnp.float32),
        pltpu.VMEM_SHARED((_N, F), jnp.float32),
    ]

    @functools.partial(pl.kernel, out_type=out, mesh=_mesh,
                       scratch_types=scratch, compiler_params=_sc_params)
    def k(yr, er, o,
          srcv, dstv, rows0, rows1, zrow, g0, g1, s0, s1, acc, ysh):
        c = lax.axis_index("c")
        s = lax.axis_index("s")
        wid = c * _NS + s

        def zfill(r, _):
            for j in range(F // 16):
                zrow[r, pl.ds(j * 16, 16)] = jnp.zeros((16,), jnp.float32)
            return 0
        lax.fori_loop(0, _ZR, zfill, 0)

        def zero_own_slice():
            def cz(j, _):
                pltpu.sync_copy(zrow, acc.at[pl.ds(s * _RPT + j * _ZR, _ZR)])
                return 0
            lax.fori_loop(0, _RPT // _ZR, cz, 0)

        zero_own_slice()
        _SR = _N // _NS
        pltpu.sync_copy(er.at[0, wid], srcv)
        pltpu.sync_copy(er.at[1, wid], dstv)
        pltpu.sync_copy(yr.at[pl.ds(s * _SR, _SR)],
                        ysh.at[pl.ds(s * _SR, _SR)])
        plsc.subcore_barrier()

        pltpu.async_copy(ysh.at[srcv.at[0]], rows0, g0)

        def it(j, _):
            i0 = 2 * j
            pltpu.make_async_copy(ysh.at[srcv.at[i0]], rows0, g0).wait()
            pltpu.async_copy(rows0, acc.at[dstv.at[i0]], s0, add=True)

            @pl.when(j > 0)
            def _s1_done():
                pltpu.make_async_copy(
                    rows1, acc.at[dstv.at[i0]], s1).wait()
            pltpu.async_copy(ysh.at[srcv.at[i0 + 1]], rows1, g1)

            pltpu.make_async_copy(ysh.at[srcv.at[i0]], rows1, g1).wait()
            pltpu.async_copy(rows1, acc.at[dstv.at[i0 + 1]], s1, add=True)
            pltpu.make_async_copy(rows0, acc.at[dstv.at[i0]], s0).wait()

            @pl.when(j < _NIT // 2 - 1)
            def _next():
                pltpu.async_copy(ysh.at[srcv.at[i0 + 2]], rows0, g0)
            return 0
        lax.fori_loop(0, _NIT // 2, it, 0)
        pltpu.make_async_copy(rows1, acc.at[dstv.at[0]], s1).wait()
        plsc.subcore_barrier()
        pltpu.sync_copy(acc.at[pl.ds(s * _RPT, _RPT)],
                        o.at[c, pl.ds(s * _RPT, _RPT)])

    return k(y, e)


def _tc_y(x, degt, W1):
    def body(x_ref, dg_ref, w_ref, y_ref):
        dinv = lax.rsqrt(dg_ref[...].sum(axis=1, keepdims=True) + 1.0)
        xw = jnp.dot(x_ref[...], w_ref[...],
                     preferred_element_type=jnp.float32)
        y_ref[...] = xw * dinv

    return pl.pallas_call(
        body, grid=(_NB,),
        in_specs=[
            pl.BlockSpec((_BR, 128), lambda i: (i, 0)),
            pl.BlockSpec((_BR, 2), lambda i: (i, 0)),
            pl.BlockSpec((128, 64), lambda i: (0, 0)),
        ],
        out_specs=pl.BlockSpec((_BR, 64), lambda i: (i, 0)),
        out_shape=jax.ShapeDtypeStruct((_N, 64), jnp.float32),
    )(x, degt, W1)


def _tc_mid(p, y, degt, W2, b1):
    def body(p_ref, y_ref, dg_ref, w_ref, b_ref, o_ref):
        dinv = lax.rsqrt(dg_ref[...].sum(axis=1, keepdims=True) + 1.0)
        pre = p_ref[0] + p_ref[1] + y_ref[...]
        h = jnp.maximum(pre * dinv + b_ref[...], 0.0)
        o_ref[...] = jnp.dot(h, w_ref[...],
                             preferred_element_type=jnp.float32) * dinv

    return pl.pallas_call(
        body, grid=(_NB,),
        in_specs=[
            pl.BlockSpec((2, _BR, 64), lambda i: (0, i, 0)),
            pl.BlockSpec((_BR, 64), lambda i: (i, 0)),
            pl.BlockSpec((_BR, 2), lambda i: (i, 0)),
            pl.BlockSpec((64, 32), lambda i: (0, 0)),
            pl.BlockSpec((1, 64), lambda i: (0, 0)),
        ],
        out_specs=pl.BlockSpec((_BR, 32), lambda i: (i, 0)),
        out_shape=jax.ShapeDtypeStruct((_N, 32), jnp.float32),
    )(p, y, degt, W2, b1)


def _tc_head(p2, y2, degt, batch3, b2, fW1, fb1, fW2, fb2):
    def body(p_ref, y_ref, dg_ref, bt_ref, b2_ref,
             w1_ref, c1_ref, w2_ref, c2_ref, o_ref, acc):
        i = pl.program_id(0)

        @pl.when(i == 0)
        def _init():
            acc[...] = jnp.zeros((_G, 64), jnp.float32)

        dinv = lax.rsqrt(dg_ref[...].sum(axis=1, keepdims=True) + 1.0)
        o2 = (p_ref[0] + p_ref[1] + y_ref[...]) * dinv + b2_ref[...]
        rows = lax.broadcasted_iota(jnp.int32, (_BR, 1), 0) + i * _BR
        valid = (rows < _N).astype(jnp.float32)
        o2 = jnp.where(rows < _N, o2, 0.0)
        b = bt_ref[0, 0, :]
        P = (b[:, None] == lax.broadcasted_iota(jnp.int32, (_BR, _G), 1)
             ).astype(jnp.float32)
        ext = jnp.concatenate(
            [o2, valid, jnp.zeros((_BR, 31), jnp.float32)], axis=1)
        acc[...] += lax.dot_general(P, ext, (((0,), (0,)), ((), ())),
                                    preferred_element_type=jnp.float32)

        @pl.when(i == _NB - 1)
        def _fin():
            a = acc[...]
            pooled = a[:, :32] / jnp.maximum(a[:, 32:33], 1.0)
            z = jnp.maximum(
                jnp.dot(pooled, w1_ref[...],
                        preferred_element_type=jnp.float32) + c1_ref[...], 0.0)
            o_ref[...] = jnp.dot(z, w2_ref[...],
                                 preferred_element_type=jnp.float32) + c2_ref[...]

    return pl.pallas_call(
        body, grid=(_NB,),
        in_specs=[
            pl.BlockSpec((2, _BR, 32), lambda i: (0, i, 0)),
            pl.BlockSpec((_BR, 32), lambda i: (i, 0)),
            pl.BlockSpec((_BR, 2), lambda i: (i, 0)),
            pl.BlockSpec((1, 1, _BR), lambda i: (i, 0, 0)),
            pl.BlockSpec((1, 32), lambda i: (0, 0)),
            pl.BlockSpec((32, 32), lambda i: (0, 0)),
            pl.BlockSpec((1, 32), lambda i: (0, 0)),
            pl.BlockSpec((32, 32), lambda i: (0, 0)),
            pl.BlockSpec((1, 32), lambda i: (0, 0)),
        ],
        out_specs=pl.BlockSpec((_G, 32), lambda i: (0, 0)),
        out_shape=jax.ShapeDtypeStruct((_G, 32), jnp.float32),
        scratch_shapes=[pltpu.VMEM((_G, 64), jnp.float32)],
    )(p2, y2, degt, batch3, b2, fW1, fb1, fW2, fb2)


def kernel(x1, edge_index1, batch1, x2, edge_index2, batch2,
           x3, edge_index3, batch3, W1, b1, W2, b2, fW1, fb1, fW2, fb2):
    npad = _EP - _E
    src_pad = (jnp.arange(npad, dtype=jnp.int32) % _N)
    dst_pad = _N + (jnp.arange(npad, dtype=jnp.int32) % (_NP - _N))
    epad = jnp.stack([src_pad, dst_pad])

    e1 = jnp.concatenate([edge_index1, epad], 1).reshape(2, _NW, _NIT, _CH)
    e2 = jnp.concatenate([edge_index2, epad], 1).reshape(2, _NW, _NIT, _CH)
    e3 = jnp.concatenate([edge_index3, epad], 1).reshape(2, _NW, _NIT, _CH)

    d1, d2, d3 = _sc_degree(e1), _sc_degree(e2), _sc_degree(e3)
    degt1, degt2, degt3 = d1.T, d2.T, d3.T

    y1 = _tc_y(x1, degt1, W1)
    y2 = _tc_y(x2, degt2, W1)
    y3 = _tc_y(x3, degt3, W1)

    p1 = _sc_scatter(64, y1, e1)
    p2 = _sc_scatter(64, y2, e2)
    p3 = _sc_scatter(64, y3, e3)

    b1r = b1.reshape(1, 64)
    z1 = _tc_mid(p1, y1, degt1, W2, b1r)
    z2 = _tc_mid(p2, y2, degt2, W2, b1r)
    z3 = _tc_mid(p3, y3, degt3, W2, b1r)

    q1 = _sc_scatter(32, z1, e1)
    q2 = _sc_scatter(32, z2, e2)
    q3 = _sc_scatter(32, z3, e3)

    pad = _NP - _N
    bt1 = jnp.pad(batch1, (0, pad), constant_values=_G).reshape(_NB, 1, _BR)
    bt2 = jnp.pad(batch2, (0, pad), constant_values=_G).reshape(_NB, 1, _BR)
    bt3 = jnp.pad(batch3, (0, pad), constant_values=_G).reshape(_NB, 1, _BR)
    b2r = b2.reshape(1, 32)
    fb1r = fb1.reshape(1, 32)
    fb2r = fb2.reshape(1, 32)

    o1 = _tc_head(q1, z1, degt1, bt1, b2r, fW1, fb1r, fW2, fb2r)
    o2 = _tc_head(q2, z2, degt2, bt2, b2r, fW1, fb1r, fW2, fb2r)
    o3 = _tc_head(q3, z3, degt3, bt3, b2r, fW1, fb1r, fW2, fb2r)
    return (o1, o2, o3)
